# baseline (speedup 1.0000x reference)
import functools

import jax
import jax.numpy as jnp
from jax import lax
from jax.experimental import pallas as pl
from jax.experimental.pallas import tpu as pltpu
from jax.experimental.pallas import tpu_sc as plsc

N = 10000
E = 160000
D = 256
DH = D // 2
EPS = 1e-5

NCORE = 2
NSUB = 16
LANES = 16
CW = 128

EDGES_PER_SUB = E // NSUB
B2 = 80
NB2 = EDGES_PER_SUB // B2
GNWB = 10
G_ROWS_WB = N // GNWB
ZCH = 200

EDGES_PER_W = E // (NCORE * NSUB)
B1 = 40
NB1 = EDGES_PER_W // B1

NODE_TILE = 1000
EDGE_TILE = 2000

@functools.lru_cache(maxsize=1)
def _sc_mesh():
    return plsc.VectorSubcoreMesh(core_axis_name="c", subcore_axis_name="s",
                                  num_cores=NCORE, num_subcores=NSUB)


def _node_mm_body(x_ref, w1_ref, b1_ref, w2_ref, b2_ref, w3_ref, b3_ref,
                  w4_ref, b4_ref, x1_ref, x2h_ref, x3_ref, x4_ref):
    xt = x_ref[...]
    x1_ref[...] = jnp.dot(xt, w1_ref[...], preferred_element_type=jnp.float32) + b1_ref[...]
    x2 = jnp.dot(xt, w2_ref[...], preferred_element_type=jnp.float32) + b2_ref[...]
    x2h_ref[0] = x2[:, :DH]
    x2h_ref[1] = x2[:, DH:]
    x3_ref[...] = jnp.dot(xt, w3_ref[...], preferred_element_type=jnp.float32) + b3_ref[...]
    x4_ref[...] = jnp.dot(xt, w4_ref[...], preferred_element_type=jnp.float32) + b4_ref[...]


def _node_mm(x, w1t, b1, w2t, b2, w3t, b3, w4t, b4):
    nw = pl.BlockSpec((D, D), lambda i: (0, 0))
    nb = pl.BlockSpec((1, D), lambda i: (0, 0))
    return pl.pallas_call(
        _node_mm_body,
        grid=(N // NODE_TILE,),
        in_specs=[pl.BlockSpec((NODE_TILE, D), lambda i: (i, 0)),
                  nw, nb, nw, nb, nw, nb, nw, nb],
        out_specs=[pl.BlockSpec((NODE_TILE, D), lambda i: (i, 0)),
                   pl.BlockSpec((NCORE, NODE_TILE, DH), lambda i: (0, i, 0)),
                   pl.BlockSpec((NODE_TILE, D), lambda i: (i, 0)),
                   pl.BlockSpec((NODE_TILE, D), lambda i: (i, 0))],
        out_shape=[jax.ShapeDtypeStruct((N, D), jnp.float32),
                   jax.ShapeDtypeStruct((NCORE, N, DH), jnp.float32),
                   jax.ShapeDtypeStruct((N, D), jnp.float32),
                   jax.ShapeDtypeStruct((N, D), jnp.float32)],
    )(x, w1t, b1, w2t, b2, w3t, b3, w4t, b4)


def _edge_mm_body(w0_ref, we_ref, be_ref, w1_ref, sig_ref):
    w0t = w0_ref[...]
    w1_ref[...] = jnp.dot(w0t, we_ref[...], preferred_element_type=jnp.float32) + be_ref[...]
    sg = jax.nn.sigmoid(w0t)
    sig_ref[0] = sg[:, :DH]
    sig_ref[1] = sg[:, DH:]


def _edge_mm(edge_attr, wet, be):
    return pl.pallas_call(
        _edge_mm_body,
        grid=(E // EDGE_TILE,),
        in_specs=[pl.BlockSpec((EDGE_TILE, D), lambda i: (i, 0)),
                  pl.BlockSpec((D, D), lambda i: (0, 0)),
                  pl.BlockSpec((1, D), lambda i: (0, 0))],
        out_specs=[pl.BlockSpec((EDGE_TILE, D), lambda i: (i, 0)),
                   pl.BlockSpec((NCORE, EDGE_TILE, DH), lambda i: (0, i, 0))],
        out_shape=[jax.ShapeDtypeStruct((E, D), jnp.float32),
                   jax.ShapeDtypeStruct((NCORE, E, DH), jnp.float32)],
    )(edge_attr, wet, be)


def _t_stats_body(a_ref, b_ref, t_ref, ssum_ref, ssq_ref):
    i = pl.program_id(0)
    t = a_ref[...] + b_ref[...]
    t_ref[...] = t
    ps = jnp.sum(t, axis=0, keepdims=True)
    pq = jnp.sum(t * t, axis=0, keepdims=True)

    @pl.when(i == 0)
    def _():
        ssum_ref[...] = ps
        ssq_ref[...] = pq

    @pl.when(i > 0)
    def _():
        ssum_ref[...] += ps
        ssq_ref[...] += pq


def _t_stats(a, b, rows, tile):
    return pl.pallas_call(
        _t_stats_body,
        grid=(rows // tile,),
        in_specs=[pl.BlockSpec((tile, D), lambda i: (i, 0)),
                  pl.BlockSpec((tile, D), lambda i: (i, 0))],
        out_specs=[pl.BlockSpec((tile, D), lambda i: (i, 0)),
                   pl.BlockSpec((1, D), lambda i: (0, 0)),
                   pl.BlockSpec((1, D), lambda i: (0, 0))],
        out_shape=[jax.ShapeDtypeStruct((rows, D), jnp.float32),
                   jax.ShapeDtypeStruct((1, D), jnp.float32),
                   jax.ShapeDtypeStruct((1, D), jnp.float32)],
    )(a, b)


def _bn_silu_res_body(t_ref, res_ref, ssum_ref, ssq_ref, g_ref, b_ref,
                      out_ref, *, rows):
    mean = ssum_ref[...] / rows
    var = ssq_ref[...] / rows - mean * mean
    inv = g_ref[...] / jnp.sqrt(var + EPS)
    bn = (t_ref[...] - mean) * inv + b_ref[...]
    out_ref[...] = res_ref[...] + bn * jax.nn.sigmoid(bn)


def _bn_silu_res(t, res, ssum, ssq, gamma, beta, rows, tile):
    return pl.pallas_call(
        functools.partial(_bn_silu_res_body, rows=float(rows)),
        grid=(rows // tile,),
        in_specs=[pl.BlockSpec((tile, D), lambda i: (i, 0)),
                  pl.BlockSpec((tile, D), lambda i: (i, 0)),
                  pl.BlockSpec((1, D), lambda i: (0, 0)),
                  pl.BlockSpec((1, D), lambda i: (0, 0)),
                  pl.BlockSpec((1, D), lambda i: (0, 0)),
                  pl.BlockSpec((1, D), lambda i: (0, 0))],
        out_specs=pl.BlockSpec((tile, D), lambda i: (i, 0)),
        out_shape=jax.ShapeDtypeStruct((rows, D), jnp.float32),
    )(t, res, ssum, ssq, gamma, beta)


def _cnt_inv_body(cnt_ref, inv_ref):
    ones = jnp.ones((CW, 1), jnp.float32)
    cw = cnt_ref[0] + cnt_ref[1]
    cnt_col = lax.dot_general(cw, ones, (((1,), (0,)), ((), ())),
                              preferred_element_type=jnp.float32)
    inv_ref[...] = 1.0 / jnp.maximum(cnt_col, 1.0)


def _cnt_inv(cnt):
    return pl.pallas_call(
        _cnt_inv_body,
        out_shape=jax.ShapeDtypeStruct((N, 1), jnp.float32),
    )(cnt)


def _tv_stats_body(x1_ref, msg_ref, inv_ref, tv_ref, ssum_ref, ssq_ref):
    i = pl.program_id(0)
    msgcat = jnp.concatenate([msg_ref[0], msg_ref[1]], axis=1)
    agg = msgcat * inv_ref[...]
    tv = x1_ref[...] + agg
    tv_ref[...] = tv
    ps = jnp.sum(tv, axis=0, keepdims=True)
    pq = jnp.sum(tv * tv, axis=0, keepdims=True)

    @pl.when(i == 0)
    def _():
        ssum_ref[...] = ps
        ssq_ref[...] = pq

    @pl.when(i > 0)
    def _():
        ssum_ref[...] += ps
        ssq_ref[...] += pq


def _tv_stats(x1, msg, inv):
    return pl.pallas_call(
        _tv_stats_body,
        grid=(N // NODE_TILE,),
        in_specs=[pl.BlockSpec((NODE_TILE, D), lambda i: (i, 0)),
                  pl.BlockSpec((NCORE, NODE_TILE, DH), lambda i: (0, i, 0)),
                  pl.BlockSpec((NODE_TILE, 1), lambda i: (i, 0))],
        out_specs=[pl.BlockSpec((NODE_TILE, D), lambda i: (i, 0)),
                   pl.BlockSpec((1, D), lambda i: (0, 0)),
                   pl.BlockSpec((1, D), lambda i: (0, 0))],
        out_shape=[jax.ShapeDtypeStruct((N, D), jnp.float32),
                   jax.ShapeDtypeStruct((1, D), jnp.float32),
                   jax.ShapeDtypeStruct((1, D), jnp.float32)],
    )(x1, msg, inv)


def _sc_gsum_body(src_hbm, dst_hbm, x3_hbm, x4_hbm, out_hbm,
                  si_v, di_v, g3_v, g4_v, sem3, sem4):
    c = lax.axis_index("c")
    s = lax.axis_index("s")
    wid = s * NCORE + c
    base0 = wid * EDGES_PER_W

    def batch_body(b, carry):
        base = base0 + b * B1
        pltpu.sync_copy(src_hbm.at[pl.ds(base, B1)], si_v.at[0])
        pltpu.sync_copy(dst_hbm.at[pl.ds(base, B1)], di_v.at[0])
        cp3 = pltpu.async_copy(x3_hbm.at[si_v.at[0]], g3_v, sem3)
        cp4 = pltpu.async_copy(x4_hbm.at[di_v.at[0]], g4_v, sem4)
        cp3.wait()
        cp4.wait()

        def row_body(r, carry2):
            def col_body(k, carry3):
                sl = pl.ds(k * LANES, LANES)
                g3_v[r, sl] = g3_v[r, sl] + g4_v[r, sl]
                return carry3
            return lax.fori_loop(0, D // LANES, col_body, carry2)

        lax.fori_loop(0, B1, row_body, 0)
        pltpu.sync_copy(g3_v, out_hbm.at[pl.ds(base, B1)])
        return carry

    lax.fori_loop(0, NB1, batch_body, 0)


def _sc_gsum(src, dst, x3, x4):
    return pl.kernel(
        _sc_gsum_body,
        out_type=jax.ShapeDtypeStruct((E, D), jnp.float32),
        mesh=_sc_mesh(),
        scratch_types=[pltpu.VMEM((1, B1), jnp.int32),
                       pltpu.VMEM((1, B1), jnp.int32),
                       pltpu.VMEM((B1, D), jnp.float32),
                       pltpu.VMEM((B1, D), jnp.float32),
                       pltpu.SemaphoreType.DMA,
                       pltpu.SemaphoreType.DMA],
    )(src, dst, x3, x4)


def _sc_gate_body(src_hbm, dst_hbm, x2h_hbm, sig_hbm, msg_hbm,
                  si_v, di_v, rows_v, sig_v, zero_v, acc_sh, sem):
    c = lax.axis_index("c")
    s = lax.axis_index("s")
    z16 = jnp.zeros((LANES,), jnp.float32)

    @pl.when(s < GNWB)
    def _():
        def zrow(r, carry):
            def zcol(k, carry2):
                zero_v[r, pl.ds(k * LANES, LANES)] = z16
                return carry2
            return lax.fori_loop(0, DH // LANES, zcol, carry)

        lax.fori_loop(0, ZCH, zrow, 0)

        def zacc(j, carry):
            pltpu.sync_copy(zero_v,
                            acc_sh.at[pl.ds(s * G_ROWS_WB + j * ZCH, ZCH)])
            return carry

        lax.fori_loop(0, G_ROWS_WB // ZCH, zacc, 0)

    plsc.subcore_barrier()

    base0 = s * EDGES_PER_SUB

    def batch_body(b, carry):
        base = base0 + b * B2
        pltpu.sync_copy(src_hbm.at[pl.ds(base, B2)], si_v)
        pltpu.sync_copy(dst_hbm.at[pl.ds(base, B2)], di_v)

        def idx_body(k, carry2):
            sl = pl.ds(k * LANES, LANES)
            di_v[sl] = di_v[sl] + c * N
            return carry2

        lax.fori_loop(0, B2 // LANES, idx_body, 0)

        pltpu.async_copy(x2h_hbm.at[di_v], rows_v, sem).wait()
        pltpu.sync_copy(sig_hbm.at[pl.ds(c * E + base, B2)], sig_v)

        def row_body(r, carry2):
            def col_body(k, carry3):
                sl = pl.ds(k * LANES, LANES)
                rows_v[r, sl] = rows_v[r, sl] * sig_v[r, sl]
                return carry3
            return lax.fori_loop(0, DH // LANES, col_body, carry2)

        lax.fori_loop(0, B2, row_body, 0)

        pltpu.sync_copy(rows_v, acc_sh.at[si_v], add=True)
        return carry

    lax.fori_loop(0, NB2, batch_body, 0)

    plsc.subcore_barrier()

    @pl.when(s < GNWB)
    def _():
        pltpu.sync_copy(
            acc_sh.at[pl.ds(s * G_ROWS_WB, G_ROWS_WB)],
            msg_hbm.at[pl.ds(c * N + s * G_ROWS_WB, G_ROWS_WB)])


CNT_NWB = 10
CNT_ROWS_WB = N // CNT_NWB
CNT_EPS_SUB = E // (NCORE * NSUB)
CNT_B = 40
CNT_NB = CNT_EPS_SUB // CNT_B


def _sc_cnt_body(src_hbm, cnt_hbm, si_v, ones_v, zc_v, cnt_sh):
    c = lax.axis_index("c")
    s = lax.axis_index("s")
    z16 = jnp.zeros((LANES,), jnp.float32)
    six16 = jnp.full((LANES,), 1.0 / CW, jnp.float32)

    def orow(r, carry):
        def ocol(k, carry2):
            ones_v[r, pl.ds(k * LANES, LANES)] = six16
            return carry2
        return lax.fori_loop(0, CW // LANES, ocol, carry)

    lax.fori_loop(0, CNT_B, orow, 0)

    def zrow(r, carry):
        def zcol(k, carry2):
            zc_v[r, pl.ds(k * LANES, LANES)] = z16
            return carry2
        return lax.fori_loop(0, CW // LANES, zcol, carry)

    lax.fori_loop(0, ZCH, zrow, 0)

    @pl.when(s < CNT_NWB)
    def _():
        def zch(j, carry):
            pltpu.sync_copy(
                zc_v, cnt_sh.at[pl.ds(s * CNT_ROWS_WB + j * ZCH, ZCH)])
            return carry
        lax.fori_loop(0, CNT_ROWS_WB // ZCH, zch, 0)

    plsc.subcore_barrier()

    base0 = c * (E // NCORE) + s * CNT_EPS_SUB

    def batch(b, carry):
        base = base0 + b * CNT_B
        pltpu.sync_copy(src_hbm.at[pl.ds(base, CNT_B)], si_v)
        pltpu.sync_copy(ones_v, cnt_sh.at[si_v], add=True)
        return carry

    lax.fori_loop(0, CNT_NB, batch, 0)

    plsc.subcore_barrier()

    @pl.when(s < CNT_NWB)
    def _():
        pltpu.sync_copy(cnt_sh.at[pl.ds(s * CNT_ROWS_WB, CNT_ROWS_WB)],
                        cnt_hbm.at[pl.ds(c * N + s * CNT_ROWS_WB, CNT_ROWS_WB)])


def _sc_cnt(src):
    return pl.kernel(
        _sc_cnt_body,
        out_type=jax.ShapeDtypeStruct((NCORE * N, CW), jnp.float32),
        mesh=_sc_mesh(),
        scratch_types=[pltpu.VMEM((CNT_B,), jnp.int32),
                       pltpu.VMEM((CNT_B, CW), jnp.float32),
                       pltpu.VMEM((ZCH, CW), jnp.float32),
                       pltpu.VMEM_SHARED((N, CW), jnp.float32)],
    )(src)


def _sc_gate_scatter(src, dst, x2h, sig):
    return pl.kernel(
        _sc_gate_body,
        out_type=jax.ShapeDtypeStruct((NCORE * N, DH), jnp.float32),
        mesh=_sc_mesh(),
        scratch_types=[pltpu.VMEM((B2,), jnp.int32),
                       pltpu.VMEM((B2,), jnp.int32),
                       pltpu.VMEM((B2, DH), jnp.float32),
                       pltpu.VMEM((B2, DH), jnp.float32),
                       pltpu.VMEM((ZCH, DH), jnp.float32),
                       pltpu.VMEM_SHARED((N, DH), jnp.float32),
                       pltpu.SemaphoreType.DMA],
    )(src, dst, x2h, sig)


def kernel(x, edge_index, edge_attr, W1, b1, W2, b2, W3, b3, W4, b4,
           We, be, gamma_v, beta_v, gamma_e, beta_e):
    r = lambda v: v.reshape(1, D)
    src = edge_index[0]
    dst = edge_index[1]
    cnt = _sc_cnt(src).reshape(NCORE, N, CW)
    x1, x2h, x3, x4 = _node_mm(x, W1.T, r(b1), W2.T, r(b2),
                               W3.T, r(b3), W4.T, r(b4))
    w1e, sig = _edge_mm(edge_attr, We.T, r(be))
    gsum = _sc_gsum(src, dst, x3, x4)
    t, esum, esq = _t_stats(w1e, gsum, E, EDGE_TILE)
    w_out = _bn_silu_res(t, edge_attr, esum, esq, r(gamma_e), r(beta_e),
                         E, EDGE_TILE)
    msg = _sc_gate_scatter(src, dst, x2h.reshape(NCORE * N, DH),
                           sig.reshape(NCORE * E, DH)).reshape(NCORE, N, DH)
    tv, nsum, nsq = _tv_stats(x1, msg, _cnt_inv(cnt))
    x_out = _bn_silu_res(tv, x, nsum, nsq, r(gamma_v), r(beta_v),
                         N, NODE_TILE)
    return (x_out, w_out)


# speedup vs baseline: 1.0624x; 1.0624x over previous
import functools

import jax
import jax.numpy as jnp
from jax import lax
from jax.experimental import pallas as pl
from jax.experimental.pallas import tpu as pltpu
from jax.experimental.pallas import tpu_sc as plsc

N = 10000
E = 160000
D = 256
DH = D // 2
EPS = 1e-5

NCORE = 2
NSUB = 16
LANES = 16
CW = 128

EDGES_PER_SUB = E // NSUB
B2 = 80
NB2 = EDGES_PER_SUB // B2
GNWB = 10
G_ROWS_WB = N // GNWB
ZCH = 200

EDGES_PER_W = E // (NCORE * NSUB)
B1 = 40
NB1 = EDGES_PER_W // B1

NODE_TILE = 1000
EDGE_TILE = 2000

@functools.lru_cache(maxsize=1)
def _sc_mesh():
    return plsc.VectorSubcoreMesh(core_axis_name="c", subcore_axis_name="s",
                                  num_cores=NCORE, num_subcores=NSUB)


def _node_mm_body(x_ref, w1_ref, b1_ref, w2_ref, b2_ref, w3_ref, b3_ref,
                  w4_ref, b4_ref, x1_ref, x2h_ref, x3_ref, x4_ref):
    xt = x_ref[...]
    x1_ref[...] = jnp.dot(xt, w1_ref[...], preferred_element_type=jnp.float32) + b1_ref[...]
    x2 = jnp.dot(xt, w2_ref[...], preferred_element_type=jnp.float32) + b2_ref[...]
    x2h_ref[0] = x2[:, :DH]
    x2h_ref[1] = x2[:, DH:]
    x3_ref[...] = jnp.dot(xt, w3_ref[...], preferred_element_type=jnp.float32) + b3_ref[...]
    x4_ref[...] = jnp.dot(xt, w4_ref[...], preferred_element_type=jnp.float32) + b4_ref[...]


def _node_mm(x, w1t, b1, w2t, b2, w3t, b3, w4t, b4):
    nw = pl.BlockSpec((D, D), lambda i: (0, 0))
    nb = pl.BlockSpec((1, D), lambda i: (0, 0))
    return pl.pallas_call(
        _node_mm_body,
        grid=(N // NODE_TILE,),
        in_specs=[pl.BlockSpec((NODE_TILE, D), lambda i: (i, 0)),
                  nw, nb, nw, nb, nw, nb, nw, nb],
        out_specs=[pl.BlockSpec((NODE_TILE, D), lambda i: (i, 0)),
                   pl.BlockSpec((NCORE, NODE_TILE, DH), lambda i: (0, i, 0)),
                   pl.BlockSpec((NODE_TILE, D), lambda i: (i, 0)),
                   pl.BlockSpec((NODE_TILE, D), lambda i: (i, 0))],
        out_shape=[jax.ShapeDtypeStruct((N, D), jnp.float32),
                   jax.ShapeDtypeStruct((NCORE, N, DH), jnp.float32),
                   jax.ShapeDtypeStruct((N, D), jnp.float32),
                   jax.ShapeDtypeStruct((N, D), jnp.float32)],
    )(x, w1t, b1, w2t, b2, w3t, b3, w4t, b4)


def _edge_mm_body(w0_ref, we_ref, be_ref, w1_ref, sig_ref):
    w0t = w0_ref[...]
    w1_ref[...] = jnp.dot(w0t, we_ref[...], preferred_element_type=jnp.float32) + be_ref[...]
    sg = jax.nn.sigmoid(w0t)
    sig_ref[0] = sg[:, :DH]
    sig_ref[1] = sg[:, DH:]


def _edge_mm(edge_attr, wet, be):
    return pl.pallas_call(
        _edge_mm_body,
        grid=(E // EDGE_TILE,),
        in_specs=[pl.BlockSpec((EDGE_TILE, D), lambda i: (i, 0)),
                  pl.BlockSpec((D, D), lambda i: (0, 0)),
                  pl.BlockSpec((1, D), lambda i: (0, 0))],
        out_specs=[pl.BlockSpec((EDGE_TILE, D), lambda i: (i, 0)),
                   pl.BlockSpec((NCORE, EDGE_TILE, DH), lambda i: (0, i, 0))],
        out_shape=[jax.ShapeDtypeStruct((E, D), jnp.float32),
                   jax.ShapeDtypeStruct((NCORE, E, DH), jnp.float32)],
    )(edge_attr, wet, be)


def _t_stats_body(a_ref, b_ref, t_ref, ssum_ref, ssq_ref):
    i = pl.program_id(0)
    t = a_ref[...] + b_ref[...]
    t_ref[...] = t
    ps = jnp.sum(t, axis=0, keepdims=True)
    pq = jnp.sum(t * t, axis=0, keepdims=True)

    @pl.when(i == 0)
    def _():
        ssum_ref[...] = ps
        ssq_ref[...] = pq

    @pl.when(i > 0)
    def _():
        ssum_ref[...] += ps
        ssq_ref[...] += pq


def _t_stats(a, b, rows, tile):
    return pl.pallas_call(
        _t_stats_body,
        grid=(rows // tile,),
        in_specs=[pl.BlockSpec((tile, D), lambda i: (i, 0)),
                  pl.BlockSpec((tile, D), lambda i: (i, 0))],
        out_specs=[pl.BlockSpec((tile, D), lambda i: (i, 0)),
                   pl.BlockSpec((1, D), lambda i: (0, 0)),
                   pl.BlockSpec((1, D), lambda i: (0, 0))],
        out_shape=[jax.ShapeDtypeStruct((rows, D), jnp.float32),
                   jax.ShapeDtypeStruct((1, D), jnp.float32),
                   jax.ShapeDtypeStruct((1, D), jnp.float32)],
    )(a, b)


def _bn_silu_res_body(t_ref, res_ref, ssum_ref, ssq_ref, g_ref, b_ref,
                      out_ref, *, rows):
    mean = ssum_ref[...] / rows
    var = ssq_ref[...] / rows - mean * mean
    inv = g_ref[...] / jnp.sqrt(var + EPS)
    bn = (t_ref[...] - mean) * inv + b_ref[...]
    out_ref[...] = res_ref[...] + bn * jax.nn.sigmoid(bn)


def _bn_silu_res(t, res, ssum, ssq, gamma, beta, rows, tile):
    return pl.pallas_call(
        functools.partial(_bn_silu_res_body, rows=float(rows)),
        grid=(rows // tile,),
        in_specs=[pl.BlockSpec((tile, D), lambda i: (i, 0)),
                  pl.BlockSpec((tile, D), lambda i: (i, 0)),
                  pl.BlockSpec((1, D), lambda i: (0, 0)),
                  pl.BlockSpec((1, D), lambda i: (0, 0)),
                  pl.BlockSpec((1, D), lambda i: (0, 0)),
                  pl.BlockSpec((1, D), lambda i: (0, 0))],
        out_specs=pl.BlockSpec((tile, D), lambda i: (i, 0)),
        out_shape=jax.ShapeDtypeStruct((rows, D), jnp.float32),
    )(t, res, ssum, ssq, gamma, beta)


def _cnt_inv_body(cnt_ref, inv_ref):
    ones = jnp.ones((CW, 1), jnp.float32)
    cw = cnt_ref[0] + cnt_ref[1]
    cnt_col = lax.dot_general(cw, ones, (((1,), (0,)), ((), ())),
                              preferred_element_type=jnp.float32)
    inv_ref[...] = 1.0 / jnp.maximum(cnt_col, 1.0)


def _cnt_inv(cnt):
    return pl.pallas_call(
        _cnt_inv_body,
        out_shape=jax.ShapeDtypeStruct((N, 1), jnp.float32),
    )(cnt)


def _tv_stats_body(x1_ref, msg_ref, inv_ref, tv_ref, ssum_ref, ssq_ref):
    i = pl.program_id(0)
    msgcat = jnp.concatenate([msg_ref[0], msg_ref[1]], axis=1)
    agg = msgcat * inv_ref[...]
    tv = x1_ref[...] + agg
    tv_ref[...] = tv
    ps = jnp.sum(tv, axis=0, keepdims=True)
    pq = jnp.sum(tv * tv, axis=0, keepdims=True)

    @pl.when(i == 0)
    def _():
        ssum_ref[...] = ps
        ssq_ref[...] = pq

    @pl.when(i > 0)
    def _():
        ssum_ref[...] += ps
        ssq_ref[...] += pq


def _tv_stats(x1, msg, inv):
    return pl.pallas_call(
        _tv_stats_body,
        grid=(N // NODE_TILE,),
        in_specs=[pl.BlockSpec((NODE_TILE, D), lambda i: (i, 0)),
                  pl.BlockSpec((NCORE, NODE_TILE, DH), lambda i: (0, i, 0)),
                  pl.BlockSpec((NODE_TILE, 1), lambda i: (i, 0))],
        out_specs=[pl.BlockSpec((NODE_TILE, D), lambda i: (i, 0)),
                   pl.BlockSpec((1, D), lambda i: (0, 0)),
                   pl.BlockSpec((1, D), lambda i: (0, 0))],
        out_shape=[jax.ShapeDtypeStruct((N, D), jnp.float32),
                   jax.ShapeDtypeStruct((1, D), jnp.float32),
                   jax.ShapeDtypeStruct((1, D), jnp.float32)],
    )(x1, msg, inv)


def _sc_gsum_body(src_hbm, dst_hbm, x3_hbm, x4_hbm, out_hbm,
                  si_v, di_v, g3_v, g4_v, sem3, sem4):
    c = lax.axis_index("c")
    s = lax.axis_index("s")
    wid = s * NCORE + c
    base0 = wid * EDGES_PER_W

    pltpu.sync_copy(src_hbm.at[pl.ds(base0, EDGES_PER_W)], si_v)
    pltpu.sync_copy(dst_hbm.at[pl.ds(base0, EDGES_PER_W)], di_v)

    def batch_body(b, carry):
        off = b * B1
        cp3 = pltpu.async_copy(x3_hbm.at[si_v.at[pl.ds(off, B1)]], g3_v, sem3)
        cp4 = pltpu.async_copy(x4_hbm.at[di_v.at[pl.ds(off, B1)]], g4_v, sem4)
        cp3.wait()
        cp4.wait()

        def row_body(r, carry2):
            def col_body(k, carry3):
                sl = pl.ds(k * LANES, LANES)
                g3_v[r, sl] = g3_v[r, sl] + g4_v[r, sl]
                return carry3
            return lax.fori_loop(0, D // LANES, col_body, carry2)

        lax.fori_loop(0, B1, row_body, 0)
        pltpu.sync_copy(g3_v, out_hbm.at[pl.ds(base0 + off, B1)])
        return carry

    lax.fori_loop(0, NB1, batch_body, 0)


def _sc_gsum(src, dst, x3, x4):
    return pl.kernel(
        _sc_gsum_body,
        out_type=jax.ShapeDtypeStruct((E, D), jnp.float32),
        mesh=_sc_mesh(),
        scratch_types=[pltpu.VMEM((EDGES_PER_W,), jnp.int32),
                       pltpu.VMEM((EDGES_PER_W,), jnp.int32),
                       pltpu.VMEM((B1, D), jnp.float32),
                       pltpu.VMEM((B1, D), jnp.float32),
                       pltpu.SemaphoreType.DMA,
                       pltpu.SemaphoreType.DMA],
    )(src, dst, x3, x4)


def _sc_gate_body(src_hbm, dst_hbm, x2h_hbm, sig_hbm, msg_hbm,
                  si_v, di_v, rows_v, sig_v, zero_v, acc_sh, sem):
    c = lax.axis_index("c")
    s = lax.axis_index("s")
    z16 = jnp.zeros((LANES,), jnp.float32)

    @pl.when(s < GNWB)
    def _():
        def zrow(r, carry):
            def zcol(k, carry2):
                zero_v[r, pl.ds(k * LANES, LANES)] = z16
                return carry2
            return lax.fori_loop(0, DH // LANES, zcol, carry)

        lax.fori_loop(0, ZCH, zrow, 0)

        def zacc(j, carry):
            pltpu.sync_copy(zero_v,
                            acc_sh.at[pl.ds(s * G_ROWS_WB + j * ZCH, ZCH)])
            return carry

        lax.fori_loop(0, G_ROWS_WB // ZCH, zacc, 0)

    plsc.subcore_barrier()

    base0 = s * EDGES_PER_SUB

    def batch_body(b, carry):
        base = base0 + b * B2
        pltpu.sync_copy(src_hbm.at[pl.ds(base, B2)], si_v)
        pltpu.sync_copy(dst_hbm.at[pl.ds(base, B2)], di_v)

        def idx_body(k, carry2):
            sl = pl.ds(k * LANES, LANES)
            di_v[sl] = di_v[sl] + c * N
            return carry2

        lax.fori_loop(0, B2 // LANES, idx_body, 0)

        pltpu.async_copy(x2h_hbm.at[di_v], rows_v, sem).wait()
        pltpu.sync_copy(sig_hbm.at[pl.ds(c * E + base, B2)], sig_v)

        def row_body(r, carry2):
            def col_body(k, carry3):
                sl = pl.ds(k * LANES, LANES)
                rows_v[r, sl] = rows_v[r, sl] * sig_v[r, sl]
                return carry3
            return lax.fori_loop(0, DH // LANES, col_body, carry2)

        lax.fori_loop(0, B2, row_body, 0)

        pltpu.sync_copy(rows_v, acc_sh.at[si_v], add=True)
        return carry

    lax.fori_loop(0, NB2, batch_body, 0)

    plsc.subcore_barrier()

    @pl.when(s < GNWB)
    def _():
        pltpu.sync_copy(
            acc_sh.at[pl.ds(s * G_ROWS_WB, G_ROWS_WB)],
            msg_hbm.at[pl.ds(c * N + s * G_ROWS_WB, G_ROWS_WB)])


CNT_NWB = 10
CNT_ROWS_WB = N // CNT_NWB
CNT_EPS_SUB = E // (NCORE * NSUB)
CNT_B = 40
CNT_NB = CNT_EPS_SUB // CNT_B


def _sc_cnt_body(src_hbm, cnt_hbm, si_v, ones_v, zc_v, cnt_sh):
    c = lax.axis_index("c")
    s = lax.axis_index("s")
    z16 = jnp.zeros((LANES,), jnp.float32)
    six16 = jnp.full((LANES,), 1.0 / CW, jnp.float32)

    def orow(r, carry):
        def ocol(k, carry2):
            ones_v[r, pl.ds(k * LANES, LANES)] = six16
            return carry2
        return lax.fori_loop(0, CW // LANES, ocol, carry)

    lax.fori_loop(0, CNT_B, orow, 0)

    def zrow(r, carry):
        def zcol(k, carry2):
            zc_v[r, pl.ds(k * LANES, LANES)] = z16
            return carry2
        return lax.fori_loop(0, CW // LANES, zcol, carry)

    lax.fori_loop(0, ZCH, zrow, 0)

    @pl.when(s < CNT_NWB)
    def _():
        def zch(j, carry):
            pltpu.sync_copy(
                zc_v, cnt_sh.at[pl.ds(s * CNT_ROWS_WB + j * ZCH, ZCH)])
            return carry
        lax.fori_loop(0, CNT_ROWS_WB // ZCH, zch, 0)

    plsc.subcore_barrier()

    base0 = c * (E // NCORE) + s * CNT_EPS_SUB

    def batch(b, carry):
        base = base0 + b * CNT_B
        pltpu.sync_copy(src_hbm.at[pl.ds(base, CNT_B)], si_v)
        pltpu.sync_copy(ones_v, cnt_sh.at[si_v], add=True)
        return carry

    lax.fori_loop(0, CNT_NB, batch, 0)

    plsc.subcore_barrier()

    @pl.when(s < CNT_NWB)
    def _():
        pltpu.sync_copy(cnt_sh.at[pl.ds(s * CNT_ROWS_WB, CNT_ROWS_WB)],
                        cnt_hbm.at[pl.ds(c * N + s * CNT_ROWS_WB, CNT_ROWS_WB)])


def _sc_cnt(src):
    return pl.kernel(
        _sc_cnt_body,
        out_type=jax.ShapeDtypeStruct((NCORE * N, CW), jnp.float32),
        mesh=_sc_mesh(),
        scratch_types=[pltpu.VMEM((CNT_B,), jnp.int32),
                       pltpu.VMEM((CNT_B, CW), jnp.float32),
                       pltpu.VMEM((ZCH, CW), jnp.float32),
                       pltpu.VMEM_SHARED((N, CW), jnp.float32)],
    )(src)


def _sc_gate_scatter(src, dst, x2h, sig):
    return pl.kernel(
        _sc_gate_body,
        out_type=jax.ShapeDtypeStruct((NCORE * N, DH), jnp.float32),
        mesh=_sc_mesh(),
        scratch_types=[pltpu.VMEM((B2,), jnp.int32),
                       pltpu.VMEM((B2,), jnp.int32),
                       pltpu.VMEM((B2, DH), jnp.float32),
                       pltpu.VMEM((B2, DH), jnp.float32),
                       pltpu.VMEM((ZCH, DH), jnp.float32),
                       pltpu.VMEM_SHARED((N, DH), jnp.float32),
                       pltpu.SemaphoreType.DMA],
    )(src, dst, x2h, sig)


def kernel(x, edge_index, edge_attr, W1, b1, W2, b2, W3, b3, W4, b4,
           We, be, gamma_v, beta_v, gamma_e, beta_e):
    r = lambda v: v.reshape(1, D)
    src = edge_index[0]
    dst = edge_index[1]
    cnt = _sc_cnt(src).reshape(NCORE, N, CW)
    x1, x2h, x3, x4 = _node_mm(x, W1.T, r(b1), W2.T, r(b2),
                               W3.T, r(b3), W4.T, r(b4))
    w1e, sig = _edge_mm(edge_attr, We.T, r(be))
    gsum = _sc_gsum(src, dst, x3, x4)
    t, esum, esq = _t_stats(w1e, gsum, E, EDGE_TILE)
    w_out = _bn_silu_res(t, edge_attr, esum, esq, r(gamma_e), r(beta_e),
                         E, EDGE_TILE)
    msg = _sc_gate_scatter(src, dst, x2h.reshape(NCORE * N, DH),
                           sig.reshape(NCORE * E, DH)).reshape(NCORE, N, DH)
    tv, nsum, nsq = _tv_stats(x1, msg, _cnt_inv(cnt))
    x_out = _bn_silu_res(tv, x, nsum, nsq, r(gamma_v), r(beta_v),
                         N, NODE_TILE)
    return (x_out, w_out)


# speedup vs baseline: 1.1255x; 1.0593x over previous
import functools

import jax
import jax.numpy as jnp
from jax import lax
from jax.experimental import pallas as pl
from jax.experimental.pallas import tpu as pltpu
from jax.experimental.pallas import tpu_sc as plsc

N = 10000
E = 160000
D = 256
DH = D // 2
EPS = 1e-5

NCORE = 2
NSUB = 16
LANES = 16
CW = 128

EDGES_PER_SUB = E // NSUB
B2 = 80
GCHUNK = 2000
GNWB = 10
G_ROWS_WB = N // GNWB
ZCH = 200

EDGES_PER_W = E // (NCORE * NSUB)
B1 = 40
NB1 = EDGES_PER_W // B1

NODE_TILE = 1000
EDGE_TILE = 2000

@functools.lru_cache(maxsize=1)
def _sc_mesh():
    return plsc.VectorSubcoreMesh(core_axis_name="c", subcore_axis_name="s",
                                  num_cores=NCORE, num_subcores=NSUB)


def _node_mm_body(x_ref, w1_ref, b1_ref, w2_ref, b2_ref, w3_ref, b3_ref,
                  w4_ref, b4_ref, x1_ref, x2h_ref, x3_ref, x4_ref):
    xt = x_ref[...]
    x1_ref[...] = jnp.dot(xt, w1_ref[...], preferred_element_type=jnp.float32) + b1_ref[...]
    x2 = jnp.dot(xt, w2_ref[...], preferred_element_type=jnp.float32) + b2_ref[...]
    x2h_ref[0] = x2[:, :DH]
    x2h_ref[1] = x2[:, DH:]
    x3_ref[...] = jnp.dot(xt, w3_ref[...], preferred_element_type=jnp.float32) + b3_ref[...]
    x4_ref[...] = jnp.dot(xt, w4_ref[...], preferred_element_type=jnp.float32) + b4_ref[...]


def _node_mm(x, w1t, b1, w2t, b2, w3t, b3, w4t, b4):
    nw = pl.BlockSpec((D, D), lambda i: (0, 0))
    nb = pl.BlockSpec((1, D), lambda i: (0, 0))
    return pl.pallas_call(
        _node_mm_body,
        grid=(N // NODE_TILE,),
        in_specs=[pl.BlockSpec((NODE_TILE, D), lambda i: (i, 0)),
                  nw, nb, nw, nb, nw, nb, nw, nb],
        out_specs=[pl.BlockSpec((NODE_TILE, D), lambda i: (i, 0)),
                   pl.BlockSpec((NCORE, NODE_TILE, DH), lambda i: (0, i, 0)),
                   pl.BlockSpec((NODE_TILE, D), lambda i: (i, 0)),
                   pl.BlockSpec((NODE_TILE, D), lambda i: (i, 0))],
        out_shape=[jax.ShapeDtypeStruct((N, D), jnp.float32),
                   jax.ShapeDtypeStruct((NCORE, N, DH), jnp.float32),
                   jax.ShapeDtypeStruct((N, D), jnp.float32),
                   jax.ShapeDtypeStruct((N, D), jnp.float32)],
    )(x, w1t, b1, w2t, b2, w3t, b3, w4t, b4)


def _edge_mm_body(w0_ref, we_ref, be_ref, w1_ref, sig_ref):
    w0t = w0_ref[...]
    w1_ref[...] = jnp.dot(w0t, we_ref[...], preferred_element_type=jnp.float32) + be_ref[...]
    sg = jax.nn.sigmoid(w0t)
    sig_ref[0] = sg[:, :DH]
    sig_ref[1] = sg[:, DH:]


def _edge_mm(edge_attr, wet, be):
    return pl.pallas_call(
        _edge_mm_body,
        grid=(E // EDGE_TILE,),
        in_specs=[pl.BlockSpec((EDGE_TILE, D), lambda i: (i, 0)),
                  pl.BlockSpec((D, D), lambda i: (0, 0)),
                  pl.BlockSpec((1, D), lambda i: (0, 0))],
        out_specs=[pl.BlockSpec((EDGE_TILE, D), lambda i: (i, 0)),
                   pl.BlockSpec((NCORE, EDGE_TILE, DH), lambda i: (0, i, 0))],
        out_shape=[jax.ShapeDtypeStruct((E, D), jnp.float32),
                   jax.ShapeDtypeStruct((NCORE, E, DH), jnp.float32)],
    )(edge_attr, wet, be)


def _t_stats_body(a_ref, b_ref, t_ref, ssum_ref, ssq_ref):
    i = pl.program_id(0)
    t = a_ref[...] + b_ref[...]
    t_ref[...] = t
    ps = jnp.sum(t, axis=0, keepdims=True)
    pq = jnp.sum(t * t, axis=0, keepdims=True)

    @pl.when(i == 0)
    def _():
        ssum_ref[...] = ps
        ssq_ref[...] = pq

    @pl.when(i > 0)
    def _():
        ssum_ref[...] += ps
        ssq_ref[...] += pq


def _t_stats(a, b, rows, tile):
    return pl.pallas_call(
        _t_stats_body,
        grid=(rows // tile,),
        in_specs=[pl.BlockSpec((tile, D), lambda i: (i, 0)),
                  pl.BlockSpec((tile, D), lambda i: (i, 0))],
        out_specs=[pl.BlockSpec((tile, D), lambda i: (i, 0)),
                   pl.BlockSpec((1, D), lambda i: (0, 0)),
                   pl.BlockSpec((1, D), lambda i: (0, 0))],
        out_shape=[jax.ShapeDtypeStruct((rows, D), jnp.float32),
                   jax.ShapeDtypeStruct((1, D), jnp.float32),
                   jax.ShapeDtypeStruct((1, D), jnp.float32)],
    )(a, b)


def _bn_silu_res_body(t_ref, res_ref, ssum_ref, ssq_ref, g_ref, b_ref,
                      out_ref, *, rows):
    mean = ssum_ref[...] / rows
    var = ssq_ref[...] / rows - mean * mean
    inv = g_ref[...] / jnp.sqrt(var + EPS)
    bn = (t_ref[...] - mean) * inv + b_ref[...]
    out_ref[...] = res_ref[...] + bn * jax.nn.sigmoid(bn)


def _bn_silu_res(t, res, ssum, ssq, gamma, beta, rows, tile):
    return pl.pallas_call(
        functools.partial(_bn_silu_res_body, rows=float(rows)),
        grid=(rows // tile,),
        in_specs=[pl.BlockSpec((tile, D), lambda i: (i, 0)),
                  pl.BlockSpec((tile, D), lambda i: (i, 0)),
                  pl.BlockSpec((1, D), lambda i: (0, 0)),
                  pl.BlockSpec((1, D), lambda i: (0, 0)),
                  pl.BlockSpec((1, D), lambda i: (0, 0)),
                  pl.BlockSpec((1, D), lambda i: (0, 0))],
        out_specs=pl.BlockSpec((tile, D), lambda i: (i, 0)),
        out_shape=jax.ShapeDtypeStruct((rows, D), jnp.float32),
    )(t, res, ssum, ssq, gamma, beta)


def _cnt_inv_body(cnt_ref, inv_ref):
    ones = jnp.ones((CW, 1), jnp.float32)
    cw = cnt_ref[0] + cnt_ref[1]
    cnt_col = lax.dot_general(cw, ones, (((1,), (0,)), ((), ())),
                              preferred_element_type=jnp.float32)
    inv_ref[...] = 1.0 / jnp.maximum(cnt_col, 1.0)


def _cnt_inv(cnt):
    return pl.pallas_call(
        _cnt_inv_body,
        out_shape=jax.ShapeDtypeStruct((N, 1), jnp.float32),
    )(cnt)


def _tv_stats_body(x1_ref, msg_ref, inv_ref, tv_ref, ssum_ref, ssq_ref):
    i = pl.program_id(0)
    msgcat = jnp.concatenate([msg_ref[0], msg_ref[1]], axis=1)
    agg = msgcat * inv_ref[...]
    tv = x1_ref[...] + agg
    tv_ref[...] = tv
    ps = jnp.sum(tv, axis=0, keepdims=True)
    pq = jnp.sum(tv * tv, axis=0, keepdims=True)

    @pl.when(i == 0)
    def _():
        ssum_ref[...] = ps
        ssq_ref[...] = pq

    @pl.when(i > 0)
    def _():
        ssum_ref[...] += ps
        ssq_ref[...] += pq


def _tv_stats(x1, msg, inv):
    return pl.pallas_call(
        _tv_stats_body,
        grid=(N // NODE_TILE,),
        in_specs=[pl.BlockSpec((NODE_TILE, D), lambda i: (i, 0)),
                  pl.BlockSpec((NCORE, NODE_TILE, DH), lambda i: (0, i, 0)),
                  pl.BlockSpec((NODE_TILE, 1), lambda i: (i, 0))],
        out_specs=[pl.BlockSpec((NODE_TILE, D), lambda i: (i, 0)),
                   pl.BlockSpec((1, D), lambda i: (0, 0)),
                   pl.BlockSpec((1, D), lambda i: (0, 0))],
        out_shape=[jax.ShapeDtypeStruct((N, D), jnp.float32),
                   jax.ShapeDtypeStruct((1, D), jnp.float32),
                   jax.ShapeDtypeStruct((1, D), jnp.float32)],
    )(x1, msg, inv)


def _sc_gsum_body(src_hbm, dst_hbm, x3_hbm, x4_hbm, out_hbm,
                  si_v, di_v, g3_v, g4_v, sem3, sem4):
    c = lax.axis_index("c")
    s = lax.axis_index("s")
    wid = s * NCORE + c
    base0 = wid * EDGES_PER_W

    pltpu.sync_copy(src_hbm.at[pl.ds(base0, EDGES_PER_W)], si_v)
    pltpu.sync_copy(dst_hbm.at[pl.ds(base0, EDGES_PER_W)], di_v)

    def batch_body(b, carry):
        off = b * B1
        cp3 = pltpu.async_copy(x3_hbm.at[si_v.at[pl.ds(off, B1)]], g3_v, sem3)
        cp4 = pltpu.async_copy(x4_hbm.at[di_v.at[pl.ds(off, B1)]], g4_v, sem4)
        cp3.wait()
        cp4.wait()

        def row_body(r, carry2):
            def col_body(k, carry3):
                sl = pl.ds(k * LANES, LANES)
                g3_v[r, sl] = g3_v[r, sl] + g4_v[r, sl]
                return carry3
            return lax.fori_loop(0, D // LANES, col_body, carry2)

        lax.fori_loop(0, B1, row_body, 0)
        pltpu.sync_copy(g3_v, out_hbm.at[pl.ds(base0 + off, B1)])
        return carry

    lax.fori_loop(0, NB1, batch_body, 0)


def _sc_gsum(src, dst, x3, x4):
    return pl.kernel(
        _sc_gsum_body,
        out_type=jax.ShapeDtypeStruct((E, D), jnp.float32),
        mesh=_sc_mesh(),
        scratch_types=[pltpu.VMEM((EDGES_PER_W,), jnp.int32),
                       pltpu.VMEM((EDGES_PER_W,), jnp.int32),
                       pltpu.VMEM((B1, D), jnp.float32),
                       pltpu.VMEM((B1, D), jnp.float32),
                       pltpu.SemaphoreType.DMA,
                       pltpu.SemaphoreType.DMA],
    )(src, dst, x3, x4)


def _sc_gate_body(src_hbm, dst_hbm, x2h_hbm, sig_hbm, msg_hbm,
                  si_v, di_v, rows_v, sig_v, zero_v, acc_sh, sem):
    c = lax.axis_index("c")
    s = lax.axis_index("s")
    z16 = jnp.zeros((LANES,), jnp.float32)

    @pl.when(s < GNWB)
    def _():
        def zrow(r, carry):
            def zcol(k, carry2):
                zero_v[r, pl.ds(k * LANES, LANES)] = z16
                return carry2
            return lax.fori_loop(0, DH // LANES, zcol, carry)

        lax.fori_loop(0, ZCH, zrow, 0)

        def zacc(j, carry):
            pltpu.sync_copy(zero_v,
                            acc_sh.at[pl.ds(s * G_ROWS_WB + j * ZCH, ZCH)])
            return carry

        lax.fori_loop(0, G_ROWS_WB // ZCH, zacc, 0)

    plsc.subcore_barrier()

    base0 = s * EDGES_PER_SUB

    def chunk_body(ch, carry):
        cbase = base0 + ch * GCHUNK
        pltpu.sync_copy(src_hbm.at[pl.ds(cbase, GCHUNK)], si_v)
        pltpu.sync_copy(dst_hbm.at[pl.ds(cbase, GCHUNK)], di_v)

        def idx_body(k, carry2):
            sl = pl.ds(k * LANES, LANES)
            di_v[sl] = di_v[sl] + c * N
            return carry2

        lax.fori_loop(0, GCHUNK // LANES, idx_body, 0)

        def batch_body(b, carry2):
            off = b * B2
            pltpu.async_copy(x2h_hbm.at[di_v.at[pl.ds(off, B2)]],
                             rows_v, sem).wait()
            pltpu.sync_copy(sig_hbm.at[pl.ds(c * E + cbase + off, B2)], sig_v)

            def row_body(r, carry3):
                def col_body(k, carry4):
                    sl = pl.ds(k * LANES, LANES)
                    rows_v[r, sl] = rows_v[r, sl] * sig_v[r, sl]
                    return carry4
                return lax.fori_loop(0, DH // LANES, col_body, carry3)

            lax.fori_loop(0, B2, row_body, 0)

            pltpu.sync_copy(rows_v, acc_sh.at[si_v.at[pl.ds(off, B2)]],
                            add=True)
            return carry2

        lax.fori_loop(0, GCHUNK // B2, batch_body, 0)
        return carry

    lax.fori_loop(0, EDGES_PER_SUB // GCHUNK, chunk_body, 0)

    plsc.subcore_barrier()

    @pl.when(s < GNWB)
    def _():
        pltpu.sync_copy(
            acc_sh.at[pl.ds(s * G_ROWS_WB, G_ROWS_WB)],
            msg_hbm.at[pl.ds(c * N + s * G_ROWS_WB, G_ROWS_WB)])


CNT_NWB = 10
CNT_ROWS_WB = N // CNT_NWB
CNT_EPS_SUB = E // (NCORE * NSUB)
CNT_B = 40
CNT_NB = CNT_EPS_SUB // CNT_B


def _sc_cnt_body(src_hbm, cnt_hbm, si_v, ones_v, zc_v, cnt_sh):
    c = lax.axis_index("c")
    s = lax.axis_index("s")
    z16 = jnp.zeros((LANES,), jnp.float32)
    six16 = jnp.full((LANES,), 1.0 / CW, jnp.float32)

    def orow(r, carry):
        def ocol(k, carry2):
            ones_v[r, pl.ds(k * LANES, LANES)] = six16
            return carry2
        return lax.fori_loop(0, CW // LANES, ocol, carry)

    lax.fori_loop(0, CNT_B, orow, 0)

    def zrow(r, carry):
        def zcol(k, carry2):
            zc_v[r, pl.ds(k * LANES, LANES)] = z16
            return carry2
        return lax.fori_loop(0, CW // LANES, zcol, carry)

    lax.fori_loop(0, ZCH, zrow, 0)

    @pl.when(s < CNT_NWB)
    def _():
        def zch(j, carry):
            pltpu.sync_copy(
                zc_v, cnt_sh.at[pl.ds(s * CNT_ROWS_WB + j * ZCH, ZCH)])
            return carry
        lax.fori_loop(0, CNT_ROWS_WB // ZCH, zch, 0)

    plsc.subcore_barrier()

    base0 = c * (E // NCORE) + s * CNT_EPS_SUB

    def batch(b, carry):
        base = base0 + b * CNT_B
        pltpu.sync_copy(src_hbm.at[pl.ds(base, CNT_B)], si_v)
        pltpu.sync_copy(ones_v, cnt_sh.at[si_v], add=True)
        return carry

    lax.fori_loop(0, CNT_NB, batch, 0)

    plsc.subcore_barrier()

    @pl.when(s < CNT_NWB)
    def _():
        pltpu.sync_copy(cnt_sh.at[pl.ds(s * CNT_ROWS_WB, CNT_ROWS_WB)],
                        cnt_hbm.at[pl.ds(c * N + s * CNT_ROWS_WB, CNT_ROWS_WB)])


def _sc_cnt(src):
    return pl.kernel(
        _sc_cnt_body,
        out_type=jax.ShapeDtypeStruct((NCORE * N, CW), jnp.float32),
        mesh=_sc_mesh(),
        scratch_types=[pltpu.VMEM((CNT_B,), jnp.int32),
                       pltpu.VMEM((CNT_B, CW), jnp.float32),
                       pltpu.VMEM((ZCH, CW), jnp.float32),
                       pltpu.VMEM_SHARED((N, CW), jnp.float32)],
    )(src)


def _sc_gate_scatter(src, dst, x2h, sig):
    return pl.kernel(
        _sc_gate_body,
        out_type=jax.ShapeDtypeStruct((NCORE * N, DH), jnp.float32),
        mesh=_sc_mesh(),
        scratch_types=[pltpu.VMEM((GCHUNK,), jnp.int32),
                       pltpu.VMEM((GCHUNK,), jnp.int32),
                       pltpu.VMEM((B2, DH), jnp.float32),
                       pltpu.VMEM((B2, DH), jnp.float32),
                       pltpu.VMEM((ZCH, DH), jnp.float32),
                       pltpu.VMEM_SHARED((N, DH), jnp.float32),
                       pltpu.SemaphoreType.DMA],
    )(src, dst, x2h, sig)


def kernel(x, edge_index, edge_attr, W1, b1, W2, b2, W3, b3, W4, b4,
           We, be, gamma_v, beta_v, gamma_e, beta_e):
    r = lambda v: v.reshape(1, D)
    src = edge_index[0]
    dst = edge_index[1]
    cnt = _sc_cnt(src).reshape(NCORE, N, CW)
    x1, x2h, x3, x4 = _node_mm(x, W1.T, r(b1), W2.T, r(b2),
                               W3.T, r(b3), W4.T, r(b4))
    w1e, sig = _edge_mm(edge_attr, We.T, r(be))
    gsum = _sc_gsum(src, dst, x3, x4)
    t, esum, esq = _t_stats(w1e, gsum, E, EDGE_TILE)
    w_out = _bn_silu_res(t, edge_attr, esum, esq, r(gamma_e), r(beta_e),
                         E, EDGE_TILE)
    msg = _sc_gate_scatter(src, dst, x2h.reshape(NCORE * N, DH),
                           sig.reshape(NCORE * E, DH)).reshape(NCORE, N, DH)
    tv, nsum, nsq = _tv_stats(x1, msg, _cnt_inv(cnt))
    x_out = _bn_silu_res(tv, x, nsum, nsq, r(gamma_v), r(beta_v),
                         N, NODE_TILE)
    return (x_out, w_out)
